# Initial kernel scaffold; baseline (speedup 1.0000x reference)
import jax
import jax.numpy as jnp
from jax.experimental import pallas as pl


def kernel(inputs, table):
    raise NotImplementedError("write your pallas kernel here")


# speedup vs baseline: 3.3432x; 3.3432x over previous
import functools

import jax
import jax.numpy as jnp
from jax import lax
from jax.experimental import pallas as pl
from jax.experimental.pallas import tpu as pltpu
from jax.experimental.pallas import tpu_sc as plsc

VOCAB = 100000
EMBED = 128
BATCH = 4096
HIST = 50

_info = plsc.get_sparse_core_info()
NC, NS = _info.num_cores, _info.num_subcores
NW = NC * NS

N = BATCH * HIST
B_PER_W = N // NW
CH = 128
NCH = B_PER_W // CH
NBUF = 5
NGROUPS = NCH // NBUF


def _gather_kernel(table_hbm, idx_hbm, out_hbm, idx_v, rows_v, gsem, osem):
    wid = lax.axis_index("s") * NC + lax.axis_index("c")
    base = wid * B_PER_W

    pltpu.sync_copy(idx_hbm.at[wid], idx_v)

    def start_gather(j, b):
        pltpu.async_copy(table_hbm.at[idx_v.at[j]], rows_v.at[b], gsem.at[b])

    def wait_gather(j, b):
        pltpu.make_async_copy(
            table_hbm.at[idx_v.at[j]], rows_v.at[b], gsem.at[b]
        ).wait()

    def start_out(j, b):
        pltpu.async_copy(
            rows_v.at[b], out_hbm.at[pl.ds(base + j * CH, CH)], osem.at[b]
        )

    def wait_out(j, b):
        pltpu.make_async_copy(
            rows_v.at[b], out_hbm.at[pl.ds(base + j * CH, CH)], osem.at[b]
        ).wait()

    for b in range(NBUF):
        start_gather(b, b)

    def group_body(g, issue_next):
        for b in range(NBUF):
            j = g * NBUF + b
            wait_gather(j, b)
            start_out(j, b)
            if issue_next:
                wait_out(j, b)
                start_gather(j + NBUF, b)

    lax.fori_loop(
        0,
        NGROUPS - 1,
        lambda g, c: (group_body(g, True), c)[1],
        0,
        unroll=False,
    )
    group_body(NGROUPS - 1, False)

    for b in range(NBUF):
        wait_out((NGROUPS - 1) * NBUF + b, b)


@jax.jit
def _embedding_gather(table, idx3):
    mesh = plsc.VectorSubcoreMesh(core_axis_name="c", subcore_axis_name="s")
    run = functools.partial(
        pl.kernel,
        mesh=mesh,
        out_type=jax.ShapeDtypeStruct((N, EMBED), jnp.float32),
        scratch_types=[
            pltpu.VMEM((NCH, CH), jnp.int32),
            pltpu.VMEM((NBUF, CH, EMBED), jnp.float32),
            pltpu.SemaphoreType.DMA((NBUF,)),
            pltpu.SemaphoreType.DMA((NBUF,)),
        ],
    )(_gather_kernel)
    return run(table, idx3)


def kernel(inputs, table):
    idx3 = inputs.astype(jnp.int32).reshape(NW, NCH, CH)
    out = _embedding_gather(table, idx3)
    return out.reshape(BATCH, HIST, EMBED)
